# baseline (speedup 1.0000x reference)
import math
import functools

import jax
import jax.numpy as jnp
from jax import lax
from jax.experimental import pallas as pl
from jax.experimental.pallas import tpu as pltpu
from jax.experimental.pallas import tpu_sc as plsc

NUM_EMBEDDINGS = 1000000
EMBED_DIM = 64
SEQ = 200
BATCH = 4096
PITCH = SEQ + 1

NC = 2
NS = 16
NW = NC * NS

B_TOTAL = BATCH * SEQ
SEQS_PER_W = BATCH // NW
IDX_W = 100
IDX_ROWS_PER_SEQ = SEQ // IDX_W


def _pe_table():
    position = jnp.arange(0, SEQ, dtype=jnp.float32)[:, None]
    div_term = jnp.exp(
        jnp.arange(0, EMBED_DIM, 2, dtype=jnp.float32)
        * (-(math.log(10000.0) / EMBED_DIM)))
    pe = jnp.zeros((SEQ, EMBED_DIM), dtype=jnp.float32)
    pe = pe.at[:, 0::2].set(jnp.sin(position * div_term))
    pe = pe.at[:, 1::2].set(jnp.cos(position * div_term))
    return pe


def _sc_gather_add_t(x2d, emb_weight, pe):
    mesh = plsc.VectorSubcoreMesh(core_axis_name="c", subcore_axis_name="s")

    @functools.partial(
        pl.kernel,
        out_type=jax.ShapeDtypeStruct((BATCH, EMBED_DIM, PITCH), jnp.float32),
        mesh=mesh,
        scratch_types=[
            pltpu.VMEM((2 * SEQS_PER_W, IDX_W), jnp.int32),
            pltpu.VMEM((SEQ, EMBED_DIM), jnp.float32),
            pltpu.VMEM((SEQ, EMBED_DIM), jnp.float32),
            pltpu.VMEM((SEQ, EMBED_DIM), jnp.float32),
            pltpu.VMEM((EMBED_DIM, PITCH), jnp.float32),
            pltpu.VMEM((EMBED_DIM, PITCH), jnp.float32),
            pltpu.SemaphoreType.DMA,
            pltpu.SemaphoreType.DMA,
            pltpu.SemaphoreType.DMA,
            pltpu.SemaphoreType.DMA,
        ],
        compiler_params=pltpu.CompilerParams(
            use_tc_tiling_on_sc=False, needs_layout_passes=False),
    )
    def k(x_hbm, tab_hbm, pe_hbm, out_hbm,
          idx_v, pe_v, bufg0, bufg1, buft0, buft1, gs0, gs1, os0, os1):
        cid = lax.axis_index("c")
        sid = lax.axis_index("s")
        wid = sid * NC + cid
        bufg = (bufg0, bufg1)
        buft = (buft0, buft1)
        gsem = (gs0, gs1)
        osem = (os0, os1)

        irow0 = pl.multiple_of(wid * (2 * SEQS_PER_W), 8)
        pltpu.sync_copy(x_hbm.at[pl.ds(irow0, 2 * SEQS_PER_W)], idx_v)
        pltpu.sync_copy(pe_hbm, pe_v)

        def g_pair(t, p):
            return (
                (tab_hbm.at[idx_v.at[2 * t]],
                 bufg[p].at[pl.ds(0, IDX_W)], gsem[p]),
                (tab_hbm.at[idx_v.at[2 * t + 1]],
                 bufg[p].at[pl.ds(IDX_W, IDX_W)], gsem[p]),
            )

        def fire_gather(t, p):
            for src, dst, sem in g_pair(t, p):
                pltpu.async_copy(src, dst, sem)

        def wait_gather(t, p):
            for src, dst, sem in g_pair(t, p):
                pltpu.make_async_copy(src, dst, sem).wait()

        def o_tuple(t, p):
            b = wid * SEQS_PER_W + t
            return buft[p], out_hbm.at[b], osem[p]

        row_ids = [lax.iota(jnp.int32, 16) + 16 * c for c in range(4)]

        def compute(p):
            def rbody(r, carry):
                colv = jnp.zeros((16,), jnp.int32) + r
                for c in range(4):
                    g = bufg[p][r, pl.ds(16 * c, 16)]
                    pv = pe_v[r, pl.ds(16 * c, 16)]
                    plsc.store_scatter(buft[p], [row_ids[c], colv], g + pv)
                return carry
            lax.fori_loop(0, SEQ, rbody, 0, unroll=2)

        fire_gather(0, 0)

        def pair_body(u, carry):
            for pp in (0, 1):
                t = 2 * u + pp
                if pp == 0:
                    fire_gather(t + 1, 1)
                else:
                    @pl.when(u < SEQS_PER_W // 2 - 1)
                    def _():
                        fire_gather(t + 1, 0)
                wait_gather(t, pp)

                @pl.when(u >= 1)
                def _():
                    src, dst, sem = o_tuple(t - 2, pp)
                    pltpu.make_async_copy(src, dst, sem).wait()

                compute(pp)
                src, dst, sem = o_tuple(t, pp)
                pltpu.async_copy(src, dst, sem)
            return carry

        lax.fori_loop(0, SEQS_PER_W // 2, pair_body, 0)

        for pp in (0, 1):
            src, dst, sem = o_tuple(SEQS_PER_W - 2 + pp, pp)
            pltpu.make_async_copy(src, dst, sem).wait()

    return k(x2d, emb_weight, pe)


def kernel(x, emb_weight):
    pe = _pe_table()
    x2d = x.reshape(B_TOTAL // IDX_W, IDX_W).astype(jnp.int32)
    g = _sc_gather_add_t(x2d, emb_weight, pe)
    out = jnp.transpose(g[:, :, :SEQ], (0, 2, 1))
    pos_emb = jnp.broadcast_to(pe[None, :, :], (BATCH, SEQ, EMBED_DIM))
    return (out, pos_emb)


# speedup vs baseline: 1.4892x; 1.4892x over previous
import math
import functools

import jax
import jax.numpy as jnp
from jax import lax
from jax.experimental import pallas as pl
from jax.experimental.pallas import tpu as pltpu
from jax.experimental.pallas import tpu_sc as plsc

NUM_EMBEDDINGS = 1000000
EMBED_DIM = 64
TAB_W = 128
SEQ = 200
BATCH = 4096

NC = 2
NS = 16
NW = NC * NS

B_TOTAL = BATCH * SEQ
ROWS_PER_W = B_TOTAL // NW
IDX_W = 128
IDX_ROWS_W = ROWS_PER_W // IDX_W
STEP = 256
N_STEPS = ROWS_PER_W // STEP


def _pe_table():
    position = jnp.arange(0, SEQ, dtype=jnp.float32)[:, None]
    div_term = jnp.exp(
        jnp.arange(0, EMBED_DIM, 2, dtype=jnp.float32)
        * (-(math.log(10000.0) / EMBED_DIM)))
    pe = jnp.zeros((SEQ, EMBED_DIM), dtype=jnp.float32)
    pe = pe.at[:, 0::2].set(jnp.sin(position * div_term))
    pe = pe.at[:, 1::2].set(jnp.cos(position * div_term))
    return pe


def _sc_gather_add(x2d, tablep, pe):
    mesh = plsc.VectorSubcoreMesh(core_axis_name="c", subcore_axis_name="s")

    @functools.partial(
        pl.kernel,
        out_type=jax.ShapeDtypeStruct((B_TOTAL, TAB_W), jnp.float32),
        mesh=mesh,
        scratch_types=[
            pltpu.VMEM((IDX_ROWS_W, IDX_W), jnp.int32),
            pltpu.VMEM((SEQ, EMBED_DIM), jnp.float32),
            pltpu.VMEM((STEP, TAB_W), jnp.float32),
            pltpu.VMEM((STEP, TAB_W), jnp.float32),
            pltpu.SemaphoreType.DMA,
            pltpu.SemaphoreType.DMA,
            pltpu.SemaphoreType.DMA,
            pltpu.SemaphoreType.DMA,
        ],
        compiler_params=pltpu.CompilerParams(
            use_tc_tiling_on_sc=True, needs_layout_passes=False),
    )
    def k(x_hbm, tab_hbm, pe_hbm, out_hbm,
          idx_v, pe_v, bufg0, bufg1, gs0, gs1, os0, os1):
        cid = lax.axis_index("c")
        sid = lax.axis_index("s")
        wid = sid * NC + cid
        bufg = (bufg0, bufg1)
        gsem = (gs0, gs1)
        osem = (os0, os1)

        irow0 = pl.multiple_of(wid * IDX_ROWS_W, 8)
        pltpu.sync_copy(x_hbm.at[pl.ds(irow0, IDX_ROWS_W)], idx_v)
        pltpu.sync_copy(pe_hbm, pe_v)
        row0_w = wid * ROWS_PER_W

        def fire_gather(kk, p):
            for j in (0, 1):
                pltpu.async_copy(
                    tab_hbm.at[idx_v.at[2 * kk + j]],
                    bufg[p].at[pl.ds(j * IDX_W, IDX_W)],
                    gsem[p],
                )

        def wait_gather(kk, p):
            for j in (0, 1):
                pltpu.make_async_copy(
                    tab_hbm.at[idx_v.at[2 * kk + j]],
                    bufg[p].at[pl.ds(j * IDX_W, IDX_W)],
                    gsem[p],
                ).wait()

        def out_dma(kk, p):
            base = row0_w + kk * STEP
            return (bufg[p], out_hbm.at[pl.ds(base, STEP)], osem[p])

        def compute(kk, p):
            base_mod = lax.rem(kk * STEP, SEQ)

            def rbody(r, carry):
                pr = lax.rem(base_mod + r, SEQ)
                for c in range(EMBED_DIM // 16):
                    pv = pe_v[pr, pl.ds(16 * c, 16)]
                    plsc.addupdate(bufg[p].at[r, pl.ds(16 * c, 16)], pv)
                return carry
            lax.fori_loop(0, STEP, rbody, 0, unroll=2)

        def step(kk, p):
            @pl.when(kk >= 2)
            def _():
                src, dst, sem = out_dma(kk - 2, p)
                pltpu.make_async_copy(src, dst, sem).wait()

            fire_gather(kk, p)

            @pl.when(kk >= 1)
            def _():
                wait_gather(kk - 1, 1 - p)
                compute(kk - 1, 1 - p)
                src, dst, sem = out_dma(kk - 1, 1 - p)
                pltpu.async_copy(src, dst, sem)

        def pair_body(u, carry):
            for pp in (0, 1):
                step(2 * u + pp, pp)
            return carry

        lax.fori_loop(0, N_STEPS // 2, pair_body, 0)

        last = N_STEPS - 1
        wait_gather(last, last % 2)
        compute(last, last % 2)
        src, dst, sem = out_dma(last, last % 2)
        pltpu.async_copy(src, dst, sem)
        for kk in (last - 1, last):
            src, dst, sem = out_dma(kk, kk % 2)
            pltpu.make_async_copy(src, dst, sem).wait()

    return k(x2d, tablep, pe)


def kernel(x, emb_weight):
    pe = _pe_table()
    x2d = x.reshape(B_TOTAL // IDX_W, IDX_W).astype(jnp.int32)
    tablep = jnp.pad(emb_weight, ((0, 0), (0, TAB_W - EMBED_DIM)))
    e = _sc_gather_add(x2d, tablep, pe)
    out = e[:, :EMBED_DIM].reshape(BATCH, SEQ, EMBED_DIM)
    pos_emb = jnp.broadcast_to(pe[None, :, :], (BATCH, SEQ, EMBED_DIM))
    return (out, pos_emb)
